# Initial kernel scaffold; baseline (speedup 1.0000x reference)
import functools

import jax
import jax.numpy as jnp
from jax import lax
from jax.experimental import pallas as pl
from jax.experimental.pallas import tpu as pltpu
from jax.experimental.pallas import tpu_sc as plsc

_N = 10000
_DIM = 128
_E = 320000
_EU = 160000
_NLAYERS = 3
_NC, _NS, _NW = 2, 16, 32
_CH = 128
_KCH = 79
_EPAD = _NW * _KCH * _CH
_KCH_E = 40
_EUPAD = _NW * _KCH_E * _CH
_ACC = 10048
_STRIPE = _ACC // _NS

_mesh = plsc.VectorSubcoreMesh(core_axis_name="c", subcore_axis_name="s")


def _deg_body(dst_i, ones_h, zeros_h, out, acc, ones_v, didx):
    c = lax.axis_index("c")
    s = lax.axis_index("s")
    wid = c * _NS + s
    row0 = s * _STRIPE
    pltpu.sync_copy(zeros_h.at[pl.ds(row0, _STRIPE)], acc.at[pl.ds(row0, _STRIPE)])
    pltpu.sync_copy(ones_h, ones_v)
    plsc.subcore_barrier()
    base = wid * _KCH * _CH

    def step(k, carry):
        off = base + k * _CH
        pltpu.sync_copy(dst_i.at[pl.ds(off, _CH)], didx)
        pltpu.sync_copy(ones_v, acc.at[didx], add=True)
        return carry

    lax.fori_loop(0, _KCH, step, 0)
    plsc.subcore_barrier()
    pltpu.sync_copy(acc.at[pl.ds(row0, _STRIPE)], out.at[c, pl.ds(row0, _STRIPE)])


_deg_kernel = functools.partial(
    pl.kernel,
    out_type=jax.ShapeDtypeStruct((_NC, _ACC, 16), jnp.float32),
    mesh=_mesh,
    scratch_types=[
        pltpu.VMEM_SHARED((_ACC, 16), jnp.float32),
        pltpu.VMEM((_CH, 16), jnp.float32),
        pltpu.VMEM((_CH,), jnp.int32),
    ],
)(_deg_body)


def _agg_body(table, src_i, dst_i, zeros_h, out, acc, sidx, didx, rows, gsem):
    c = lax.axis_index("c")
    s = lax.axis_index("s")
    wid = c * _NS + s
    row0 = s * _STRIPE
    pltpu.sync_copy(zeros_h.at[pl.ds(row0, _STRIPE)], acc.at[pl.ds(row0, _STRIPE)])
    plsc.subcore_barrier()
    base = wid * _KCH * _CH

    def step(k, carry):
        off = base + k * _CH
        pltpu.sync_copy(src_i.at[pl.ds(off, _CH)], sidx)
        pltpu.sync_copy(dst_i.at[pl.ds(off, _CH)], didx)
        pltpu.async_copy(table.at[sidx], rows, gsem).wait()
        pltpu.sync_copy(rows, acc.at[didx], add=True)
        return carry

    lax.fori_loop(0, _KCH, step, 0)
    plsc.subcore_barrier()
    pltpu.sync_copy(acc.at[pl.ds(row0, _STRIPE)], out.at[c, pl.ds(row0, _STRIPE)])


_agg_kernel = functools.partial(
    pl.kernel,
    out_type=jax.ShapeDtypeStruct((_NC, _ACC, _DIM), jnp.float32),
    mesh=_mesh,
    scratch_types=[
        pltpu.VMEM_SHARED((_ACC, _DIM), jnp.float32),
        pltpu.VMEM((_CH,), jnp.int32),
        pltpu.VMEM((_CH,), jnp.int32),
        pltpu.VMEM((_CH, _DIM), jnp.float32),
        pltpu.SemaphoreType.DMA,
    ],
)(_agg_body)


def _egather_body(ta, tb, src_i, dst_i, outa, outb, sidx, didx, rowsa, rowsb,
                  sema, semb):
    c = lax.axis_index("c")
    s = lax.axis_index("s")
    wid = c * _NS + s
    base = wid * _KCH_E * _CH

    def step(k, carry):
        off = base + k * _CH
        pltpu.sync_copy(src_i.at[pl.ds(off, _CH)], sidx)
        pltpu.sync_copy(dst_i.at[pl.ds(off, _CH)], didx)
        cpa = pltpu.async_copy(ta.at[sidx], rowsa, sema)
        cpb = pltpu.async_copy(tb.at[didx], rowsb, semb)
        cpa.wait()
        cpb.wait()
        pltpu.sync_copy(rowsa, outa.at[pl.ds(off, _CH)])
        pltpu.sync_copy(rowsb, outb.at[pl.ds(off, _CH)])
        return carry

    lax.fori_loop(0, _KCH_E, step, 0)


_egather_kernel = functools.partial(
    pl.kernel,
    out_type=(
        jax.ShapeDtypeStruct((_EUPAD, _DIM), jnp.float32),
        jax.ShapeDtypeStruct((_EUPAD, _DIM), jnp.float32),
    ),
    mesh=_mesh,
    scratch_types=[
        pltpu.VMEM((_CH,), jnp.int32),
        pltpu.VMEM((_CH,), jnp.int32),
        pltpu.VMEM((_CH, _DIM), jnp.float32),
        pltpu.VMEM((_CH, _DIM), jnp.float32),
        pltpu.SemaphoreType.DMA,
        pltpu.SemaphoreType.DMA,
    ],
)(_egather_body)


_BR = 2000


def _mm_scale_body(h_ref, w_ref, degp_ref, out_ref):
    dinv = lax.rsqrt(degp_ref[0, :, 0:1] + degp_ref[1, :, 0:1] + 1.0)
    hw = lax.dot_general(h_ref[...], w_ref[...], (((1,), (1,)), ((), ())),
                         preferred_element_type=jnp.float32)
    out_ref[...] = hw * dinv


def _mm_scale(h, w, degp):
    return pl.pallas_call(
        _mm_scale_body,
        grid=(_N // _BR,),
        in_specs=[
            pl.BlockSpec((_BR, _DIM), lambda i: (i, 0)),
            pl.BlockSpec((_DIM, _DIM), lambda i: (0, 0)),
            pl.BlockSpec((_NC, _BR, 16), lambda i: (0, i, 0)),
        ],
        out_specs=pl.BlockSpec((_BR, _DIM), lambda i: (i, 0)),
        out_shape=jax.ShapeDtypeStruct((_N, _DIM), jnp.float32),
    )(h, w, degp)


def _ln_res_body(h_ref, hwp_ref, sp_ref, degp_ref, cb_ref, lw_ref, lb_ref,
                 out_ref):
    dinv = lax.rsqrt(degp_ref[0, :, 0:1] + degp_ref[1, :, 0:1] + 1.0)
    seg = sp_ref[0] + sp_ref[1] + hwp_ref[...]
    u = h_ref[...] + dinv * seg + cb_ref[...]
    mu = jnp.mean(u, axis=-1, keepdims=True)
    d = u - mu
    var = jnp.mean(d * d, axis=-1, keepdims=True)
    out_ref[...] = d * lax.rsqrt(var + 1e-5) * lw_ref[...] + lb_ref[...]


def _ln_res(h, hwp, sp, degp, cb, lw, lb):
    return pl.pallas_call(
        _ln_res_body,
        grid=(_N // _BR,),
        in_specs=[
            pl.BlockSpec((_BR, _DIM), lambda i: (i, 0)),
            pl.BlockSpec((_BR, _DIM), lambda i: (i, 0)),
            pl.BlockSpec((_NC, _BR, _DIM), lambda i: (0, i, 0)),
            pl.BlockSpec((_NC, _BR, 16), lambda i: (0, i, 0)),
            pl.BlockSpec((1, _DIM), lambda i: (0, 0)),
            pl.BlockSpec((1, _DIM), lambda i: (0, 0)),
            pl.BlockSpec((1, _DIM), lambda i: (0, 0)),
        ],
        out_specs=pl.BlockSpec((_BR, _DIM), lambda i: (i, 0)),
        out_shape=jax.ShapeDtypeStruct((_N, _DIM), jnp.float32),
    )(h, hwp, sp, degp, cb, lw, lb)


def _ab_body(h_ref, wa_ref, wb_ref, outa_ref, outb_ref):
    h = h_ref[...]
    outa_ref[...] = jnp.dot(h, wa_ref[...], preferred_element_type=jnp.float32)
    outb_ref[...] = jnp.dot(h, wb_ref[...], preferred_element_type=jnp.float32)


def _ab_proj(h, wa, wb):
    return pl.pallas_call(
        _ab_body,
        grid=(_N // _BR,),
        in_specs=[
            pl.BlockSpec((_BR, _DIM), lambda i: (i, 0)),
            pl.BlockSpec((_DIM, _DIM), lambda i: (0, 0)),
            pl.BlockSpec((_DIM, _DIM), lambda i: (0, 0)),
        ],
        out_specs=(
            pl.BlockSpec((_BR, _DIM), lambda i: (i, 0)),
            pl.BlockSpec((_BR, _DIM), lambda i: (i, 0)),
        ),
        out_shape=(
            jax.ShapeDtypeStruct((_N, _DIM), jnp.float32),
            jax.ShapeDtypeStruct((_N, _DIM), jnp.float32),
        ),
    )(h, wa, wb)


_BE = 2000


def _emlp_body(ga_ref, gb_ref, ef_ref, wc_ref, w2_ref, b1_ref, b2_ref,
               out_ref):
    t = (ga_ref[...] + gb_ref[...]
         + jnp.dot(ef_ref[...], wc_ref[...], preferred_element_type=jnp.float32)
         + b1_ref[...])
    hid = jnp.maximum(t, 0.0)
    out_ref[...] = (jnp.dot(hid, w2_ref[...], preferred_element_type=jnp.float32)
                    + b2_ref[...])


def _emlp(ga, gb, ef, wc, w2, b1, b2):
    return pl.pallas_call(
        _emlp_body,
        grid=(_EU // _BE,),
        in_specs=[
            pl.BlockSpec((_BE, _DIM), lambda i: (i, 0)),
            pl.BlockSpec((_BE, _DIM), lambda i: (i, 0)),
            pl.BlockSpec((_BE, 16), lambda i: (i, 0)),
            pl.BlockSpec((16, _DIM), lambda i: (0, 0)),
            pl.BlockSpec((_DIM, _DIM), lambda i: (0, 0)),
            pl.BlockSpec((1, _DIM), lambda i: (0, 0)),
            pl.BlockSpec((1, _DIM), lambda i: (0, 0)),
        ],
        out_specs=pl.BlockSpec((_BE, _DIM), lambda i: (i, 0)),
        out_shape=jax.ShapeDtypeStruct((_EU, _DIM), jnp.float32),
    )(ga, gb, ef, wc, w2, b1, b2)


def kernel(x, edge_index, edge_features, convW, convB, lnW, lnB, eW1, eB1,
           eW2, eB2):
    src_all = edge_index[0]
    dst_all = edge_index[1]
    pad_e = _EPAD - _E
    src_pad = jnp.concatenate([src_all, jnp.zeros((pad_e,), jnp.int32)])
    dst_pad = jnp.concatenate([dst_all, jnp.full((pad_e,), _N, jnp.int32)])

    zeros128 = jnp.zeros((_ACC, _DIM), jnp.float32)
    zeros16 = jnp.zeros((_ACC, 16), jnp.float32)
    ones16 = jnp.ones((_CH, 16), jnp.float32)

    degp = _deg_kernel(dst_pad, ones16, zeros16)

    h = x
    for l in range(_NLAYERS):
        hwp = _mm_scale(h, convW[l], degp)
        sp = _agg_kernel(hwp, src_pad, dst_pad, zeros128)
        sp = sp[:, :_N, :]
        h = _ln_res(h, hwp, sp, degp, convB[l].reshape(1, _DIM),
                    lnW[l].reshape(1, _DIM), lnB[l].reshape(1, _DIM))

    srcu = edge_index[0, 0::2]
    dstu = edge_index[1, 0::2]
    pad_u = _EUPAD - _EU
    srcu_pad = jnp.concatenate([srcu, jnp.zeros((pad_u,), jnp.int32)])
    dstu_pad = jnp.concatenate([dstu, jnp.zeros((pad_u,), jnp.int32)])

    e_w1t = eW1.T
    wa = e_w1t[:_DIM]
    wb = e_w1t[_DIM:2 * _DIM]
    wc = e_w1t[2 * _DIM:]
    a, b = _ab_proj(h, wa, wb)
    ga, gb = _egather_kernel(a, b, srcu_pad, dstu_pad)
    edge_emb = _emlp(ga[:_EU], gb[:_EU], edge_features, wc, eW2.T,
                     eB1.reshape(1, _DIM), eB2.reshape(1, _DIM))
    return (h, edge_emb)


# speedup vs baseline: 6.0237x; 6.0237x over previous
import functools

import jax
import jax.numpy as jnp
from jax import lax
from jax.experimental import pallas as pl
from jax.experimental.pallas import tpu as pltpu
from jax.experimental.pallas import tpu_sc as plsc

_N = 10000
_DIM = 128
_E = 320000
_EU = 160000
_NLAYERS = 3
_NC, _NS, _NW = 2, 16, 32
_CH = 128
_KCH = 79
_EPAD = _NW * _KCH * _CH
_KCH_E = 40
_EUPAD = _NW * _KCH_E * _CH
_ACC = 10112
_STRIPE = _ACC // _NS

@functools.lru_cache(maxsize=None)
def _mesh():
    return plsc.VectorSubcoreMesh(core_axis_name="c", subcore_axis_name="s")


def _deg_body(dst_i, ones_h, zeros_h, out, acc, ones_v, didx):
    c = lax.axis_index("c")
    s = lax.axis_index("s")
    wid = c * _NS + s
    row0 = s * _STRIPE
    pltpu.sync_copy(zeros_h.at[pl.ds(row0, _STRIPE)], acc.at[pl.ds(row0, _STRIPE)])
    pltpu.sync_copy(ones_h, ones_v)
    plsc.subcore_barrier()
    base = wid * _KCH * _CH

    def step(k, carry):
        off = base + k * _CH
        pltpu.sync_copy(dst_i.at[pl.ds(off, _CH)], didx)
        pltpu.sync_copy(ones_v, acc.at[didx], add=True)
        return carry

    lax.fori_loop(0, _KCH, step, 0)
    plsc.subcore_barrier()
    pltpu.sync_copy(acc.at[pl.ds(row0, _STRIPE)], out.at[c, pl.ds(row0, _STRIPE)])


@functools.lru_cache(maxsize=None)
def _deg_kernel():
    return pl.kernel(
        _deg_body,
        out_type=jax.ShapeDtypeStruct((_NC, _ACC, _DIM), jnp.float32),
        mesh=_mesh(),
        scratch_types=[
            pltpu.VMEM_SHARED((_ACC, _DIM), jnp.float32),
            pltpu.VMEM((_CH, _DIM), jnp.float32),
            pltpu.VMEM((_CH,), jnp.int32),
        ],
    )


def _agg_body(table, src_i, dst_i, zeros_h, out, acc, sidx, didx, rows, gsem):
    c = lax.axis_index("c")
    s = lax.axis_index("s")
    wid = c * _NS + s
    row0 = s * _STRIPE
    pltpu.sync_copy(zeros_h.at[pl.ds(row0, _STRIPE)], acc.at[pl.ds(row0, _STRIPE)])
    plsc.subcore_barrier()
    base = wid * _KCH * _CH

    def step(k, carry):
        off = base + k * _CH
        pltpu.sync_copy(src_i.at[pl.ds(off, _CH)], sidx)
        pltpu.sync_copy(dst_i.at[pl.ds(off, _CH)], didx)
        pltpu.async_copy(table.at[sidx], rows, gsem).wait()
        pltpu.sync_copy(rows, acc.at[didx], add=True)
        return carry

    lax.fori_loop(0, _KCH, step, 0)
    plsc.subcore_barrier()
    pltpu.sync_copy(acc.at[pl.ds(row0, _STRIPE)], out.at[c, pl.ds(row0, _STRIPE)])


@functools.lru_cache(maxsize=None)
def _agg_kernel():
    return pl.kernel(
        _agg_body,
        out_type=jax.ShapeDtypeStruct((_NC, _ACC, _DIM), jnp.float32),
        mesh=_mesh(),
        scratch_types=[
            pltpu.VMEM_SHARED((_ACC, _DIM), jnp.float32),
            pltpu.VMEM((_CH,), jnp.int32),
            pltpu.VMEM((_CH,), jnp.int32),
            pltpu.VMEM((_CH, _DIM), jnp.float32),
            pltpu.SemaphoreType.DMA,
        ],
    )


def _egather_body(ta, tb, src_i, dst_i, outa, outb, sidx, didx, rowsa, rowsb,
                  sema, semb):
    c = lax.axis_index("c")
    s = lax.axis_index("s")
    wid = c * _NS + s
    base = wid * _KCH_E * _CH

    def step(k, carry):
        off = base + k * _CH
        pltpu.sync_copy(src_i.at[pl.ds(off, _CH)], sidx)
        pltpu.sync_copy(dst_i.at[pl.ds(off, _CH)], didx)
        cpa = pltpu.async_copy(ta.at[sidx], rowsa, sema)
        cpb = pltpu.async_copy(tb.at[didx], rowsb, semb)
        cpa.wait()
        cpb.wait()
        pltpu.sync_copy(rowsa, outa.at[pl.ds(off, _CH)])
        pltpu.sync_copy(rowsb, outb.at[pl.ds(off, _CH)])
        return carry

    lax.fori_loop(0, _KCH_E, step, 0)


@functools.lru_cache(maxsize=None)
def _egather_kernel():
    return pl.kernel(
        _egather_body,
        out_type=(
            jax.ShapeDtypeStruct((_EUPAD, _DIM), jnp.float32),
            jax.ShapeDtypeStruct((_EUPAD, _DIM), jnp.float32),
        ),
        mesh=_mesh(),
        scratch_types=[
            pltpu.VMEM((_CH,), jnp.int32),
            pltpu.VMEM((_CH,), jnp.int32),
            pltpu.VMEM((_CH, _DIM), jnp.float32),
            pltpu.VMEM((_CH, _DIM), jnp.float32),
            pltpu.SemaphoreType.DMA,
            pltpu.SemaphoreType.DMA,
        ],
    )


_BR = 2000


def _mm_scale_body(h_ref, w_ref, degp_ref, out_ref):
    dinv = lax.rsqrt(degp_ref[0, :, 0:1] + degp_ref[1, :, 0:1] + 1.0)
    hw = lax.dot_general(h_ref[...], w_ref[...], (((1,), (1,)), ((), ())),
                         preferred_element_type=jnp.float32)
    out_ref[...] = hw * dinv


def _mm_scale(h, w, degp):
    return pl.pallas_call(
        _mm_scale_body,
        grid=(_N // _BR,),
        in_specs=[
            pl.BlockSpec((_BR, _DIM), lambda i: (i, 0)),
            pl.BlockSpec((_DIM, _DIM), lambda i: (0, 0)),
            pl.BlockSpec((_NC, _BR, _DIM), lambda i: (0, i, 0)),
        ],
        out_specs=pl.BlockSpec((_BR, _DIM), lambda i: (i, 0)),
        out_shape=jax.ShapeDtypeStruct((_N, _DIM), jnp.float32),
    )(h, w, degp)


def _ln_res_body(h_ref, hwp_ref, sp_ref, degp_ref, cb_ref, lw_ref, lb_ref,
                 out_ref):
    dinv = lax.rsqrt(degp_ref[0, :, 0:1] + degp_ref[1, :, 0:1] + 1.0)
    seg = sp_ref[0] + sp_ref[1] + hwp_ref[...]
    u = h_ref[...] + dinv * seg + cb_ref[...]
    mu = jnp.mean(u, axis=-1, keepdims=True)
    d = u - mu
    var = jnp.mean(d * d, axis=-1, keepdims=True)
    out_ref[...] = d * lax.rsqrt(var + 1e-5) * lw_ref[...] + lb_ref[...]


def _ln_res(h, hwp, sp, degp, cb, lw, lb):
    return pl.pallas_call(
        _ln_res_body,
        grid=(_N // _BR,),
        in_specs=[
            pl.BlockSpec((_BR, _DIM), lambda i: (i, 0)),
            pl.BlockSpec((_BR, _DIM), lambda i: (i, 0)),
            pl.BlockSpec((_NC, _BR, _DIM), lambda i: (0, i, 0)),
            pl.BlockSpec((_NC, _BR, _DIM), lambda i: (0, i, 0)),
            pl.BlockSpec((1, _DIM), lambda i: (0, 0)),
            pl.BlockSpec((1, _DIM), lambda i: (0, 0)),
            pl.BlockSpec((1, _DIM), lambda i: (0, 0)),
        ],
        out_specs=pl.BlockSpec((_BR, _DIM), lambda i: (i, 0)),
        out_shape=jax.ShapeDtypeStruct((_N, _DIM), jnp.float32),
    )(h, hwp, sp, degp, cb, lw, lb)


def _ab_body(h_ref, wa_ref, wb_ref, outa_ref, outb_ref):
    h = h_ref[...]
    outa_ref[...] = jnp.dot(h, wa_ref[...], preferred_element_type=jnp.float32)
    outb_ref[...] = jnp.dot(h, wb_ref[...], preferred_element_type=jnp.float32)


def _ab_proj(h, wa, wb):
    return pl.pallas_call(
        _ab_body,
        grid=(_N // _BR,),
        in_specs=[
            pl.BlockSpec((_BR, _DIM), lambda i: (i, 0)),
            pl.BlockSpec((_DIM, _DIM), lambda i: (0, 0)),
            pl.BlockSpec((_DIM, _DIM), lambda i: (0, 0)),
        ],
        out_specs=(
            pl.BlockSpec((_BR, _DIM), lambda i: (i, 0)),
            pl.BlockSpec((_BR, _DIM), lambda i: (i, 0)),
        ),
        out_shape=(
            jax.ShapeDtypeStruct((_N, _DIM), jnp.float32),
            jax.ShapeDtypeStruct((_N, _DIM), jnp.float32),
        ),
    )(h, wa, wb)


_BE = 2000


def _emlp_body(ga_ref, gb_ref, ef_ref, wc_ref, w2_ref, b1_ref, b2_ref,
               out_ref):
    t = (ga_ref[...] + gb_ref[...]
         + jnp.dot(ef_ref[...], wc_ref[...], preferred_element_type=jnp.float32)
         + b1_ref[...])
    hid = jnp.maximum(t, 0.0)
    out_ref[...] = (jnp.dot(hid, w2_ref[...], preferred_element_type=jnp.float32)
                    + b2_ref[...])


def _emlp(ga, gb, ef, wc, w2, b1, b2):
    return pl.pallas_call(
        _emlp_body,
        grid=(_EU // _BE,),
        in_specs=[
            pl.BlockSpec((_BE, _DIM), lambda i: (i, 0)),
            pl.BlockSpec((_BE, _DIM), lambda i: (i, 0)),
            pl.BlockSpec((_BE, 16), lambda i: (i, 0)),
            pl.BlockSpec((16, _DIM), lambda i: (0, 0)),
            pl.BlockSpec((_DIM, _DIM), lambda i: (0, 0)),
            pl.BlockSpec((1, _DIM), lambda i: (0, 0)),
            pl.BlockSpec((1, _DIM), lambda i: (0, 0)),
        ],
        out_specs=pl.BlockSpec((_BE, _DIM), lambda i: (i, 0)),
        out_shape=jax.ShapeDtypeStruct((_EU, _DIM), jnp.float32),
    )(ga, gb, ef, wc, w2, b1, b2)


def kernel(x, edge_index, edge_features, convW, convB, lnW, lnB, eW1, eB1,
           eW2, eB2):
    src_all = edge_index[0]
    dst_all = edge_index[1]
    pad_e = _EPAD - _E
    src_pad = jnp.concatenate([src_all, jnp.zeros((pad_e,), jnp.int32)])
    dst_pad = jnp.concatenate([dst_all, jnp.full((pad_e,), _N, jnp.int32)])

    zeros128 = jnp.zeros((_ACC, _DIM), jnp.float32)
    ones128 = jnp.ones((_CH, _DIM), jnp.float32)

    degp = _deg_kernel()(dst_pad, ones128, zeros128)

    h = x
    for l in range(_NLAYERS):
        hwp = _mm_scale(h, convW[l], degp)
        sp = _agg_kernel()(hwp, src_pad, dst_pad, zeros128)
        sp = sp[:, :_N, :]
        h = _ln_res(h, hwp, sp, degp, convB[l].reshape(1, _DIM),
                    lnW[l].reshape(1, _DIM), lnB[l].reshape(1, _DIM))

    srcu = edge_index[0, 0::2]
    dstu = edge_index[1, 0::2]
    pad_u = _EUPAD - _EU
    srcu_pad = jnp.concatenate([srcu, jnp.zeros((pad_u,), jnp.int32)])
    dstu_pad = jnp.concatenate([dstu, jnp.zeros((pad_u,), jnp.int32)])

    e_w1t = eW1.T
    wa = e_w1t[:_DIM]
    wb = e_w1t[_DIM:2 * _DIM]
    wc = e_w1t[2 * _DIM:]
    a, b = _ab_proj(h, wa, wb)
    ga, gb = _egather_kernel()(a, b, srcu_pad, dstu_pad)
    edge_emb = _emlp(ga[:_EU], gb[:_EU], edge_features, wc, eW2.T,
                     eB1.reshape(1, _DIM), eB2.reshape(1, _DIM))
    return (h, edge_emb)
